# Initial kernel scaffold; baseline (speedup 1.0000x reference)
import jax
import jax.numpy as jnp
from jax.experimental import pallas as pl


def kernel(feat, segment_ids, num_graphs):
    raise NotImplementedError("write your pallas kernel here")


# speedup vs baseline: 1.5863x; 1.5863x over previous
import functools

import jax
import jax.numpy as jnp
from jax import lax
from jax.experimental import pallas as pl
from jax.experimental.pallas import tpu as pltpu
from jax.experimental.pallas import tpu_sc as plsc

N = 50000
D = 256
G = 256
L = 16
NW = 32
RB = 200
NBLK = N // RB
_EXTRA = NBLK % NW
_BASE_BLKS = NBLK // NW

_mesh = plsc.VectorSubcoreMesh(core_axis_name="c", subcore_axis_name="s")

_NEG = float("-inf")


@functools.partial(
    pl.kernel,
    out_type=jax.ShapeDtypeStruct((NW * G * D,), jnp.float32),
    mesh=_mesh,
    compiler_params=pltpu.CompilerParams(needs_layout_passes=False),
    scratch_types=[
        pltpu.VMEM((RB * D,), jnp.float32),
        pltpu.VMEM((G * D,), jnp.float32),
        pltpu.VMEM((256,), jnp.int32),
    ],
)
def _seg_max_partial(feat_hbm, seg_hbm, part_hbm, rowbuf, accf, segbuf):
    w = lax.axis_index("s") * 2 + lax.axis_index("c")
    iota = lax.broadcasted_iota(jnp.int32, (L,), 0)
    neg = jnp.full((L,), _NEG, jnp.float32)

    def init_body(i, carry):
        accf[pl.ds(i * L, L)] = neg
        return carry

    lax.fori_loop(0, G * D // L, init_body, 0)

    nb = jnp.where(w < _EXTRA, _BASE_BLKS + 1, _BASE_BLKS)

    def blk_body(k, carry):
        b = w + NW * k
        pltpu.sync_copy(feat_hbm.at[pl.ds(b * RB * D, RB * D)], rowbuf)
        pltpu.sync_copy(seg_hbm.at[pl.ds(b * RB, RB)], segbuf.at[pl.ds(0, RB)])

        def row_body(r, rc):
            rvec = lax.broadcast(r, (L,))
            base = plsc.load_gather(segbuf, [rvec])
            idx0 = base + iota
            off = r * D
            for c in range(D // L):
                idxc = idx0 + (c * L)
                old = plsc.load_gather(accf, [idxc])
                val = rowbuf[pl.ds(off + c * L, L)]
                plsc.store_scatter(accf, [idxc], jnp.maximum(old, val))
            return rc

        lax.fori_loop(0, RB, row_body, 0)
        return carry

    lax.fori_loop(0, nb, blk_body, 0)
    pltpu.sync_copy(accf, part_hbm.at[pl.ds(w * G * D, G * D)])


_SEG_PER_W = G // NW
_CH = _SEG_PER_W * D


@functools.partial(
    pl.kernel,
    out_type=jax.ShapeDtypeStruct((G * D,), jnp.float32),
    mesh=_mesh,
    compiler_params=pltpu.CompilerParams(needs_layout_passes=False),
    scratch_types=[
        pltpu.VMEM((_CH,), jnp.float32),
        pltpu.VMEM((_CH,), jnp.float32),
    ],
)
def _combine(part_hbm, out_hbm, buf, acc):
    w = lax.axis_index("s") * 2 + lax.axis_index("c")
    neg = jnp.full((L,), _NEG, jnp.float32)

    def init_body(i, carry):
        acc[pl.ds(i * L, L)] = neg
        return carry

    lax.fori_loop(0, _CH // L, init_body, 0)

    def t_body(t, carry):
        pltpu.sync_copy(part_hbm.at[pl.ds(t * (G * D) + w * _CH, _CH)], buf)

        def m_body(i, mc):
            sl = pl.ds(i * L, L)
            acc[sl] = jnp.maximum(acc[sl], buf[sl])
            return mc

        lax.fori_loop(0, _CH // L, m_body, 0)
        return carry

    lax.fori_loop(0, NW, t_body, 0)
    pltpu.sync_copy(acc, out_hbm.at[pl.ds(w * _CH, _CH)])


def kernel(feat, segment_ids, num_graphs):
    seg = jnp.minimum(segment_ids, num_graphs - 1).astype(jnp.int32) * jnp.int32(D)
    part = _seg_max_partial(feat.reshape(-1), seg)
    outf = _combine(part)
    return outf.reshape(G, D)


# speedup vs baseline: 1.6333x; 1.0297x over previous
import functools

import jax
import jax.numpy as jnp
from jax import lax
from jax.experimental import pallas as pl
from jax.experimental.pallas import tpu as pltpu
from jax.experimental.pallas import tpu_sc as plsc

N = 50000
D = 256
G = 256
L = 16
NW = 32
RB = 200
NBLK = N // RB
_EXTRA = NBLK % NW
_BASE_BLKS = NBLK // NW

_mesh = plsc.VectorSubcoreMesh(core_axis_name="c", subcore_axis_name="s")

_NEG = float("-inf")


@functools.partial(
    pl.kernel,
    out_type=jax.ShapeDtypeStruct((NW * G * D,), jnp.float32),
    mesh=_mesh,
    compiler_params=pltpu.CompilerParams(needs_layout_passes=False),
    scratch_types=[
        pltpu.VMEM((RB * D,), jnp.float32),
        pltpu.VMEM((G * D,), jnp.float32),
        pltpu.VMEM((256,), jnp.int32),
    ],
)
def _seg_max_partial(feat_hbm, seg_hbm, part_hbm, rowbuf, accf, segbuf):
    w = lax.axis_index("s") * 2 + lax.axis_index("c")
    iota = lax.broadcasted_iota(jnp.int32, (L,), 0)
    neg = jnp.full((L,), _NEG, jnp.float32)
    NC = D // L

    def init_body(i, carry):
        accf[pl.ds(i * L, L)] = neg
        return carry

    lax.fori_loop(0, G * D // L, init_body, 0)

    nb = jnp.where(w < _EXTRA, _BASE_BLKS + 1, _BASE_BLKS)

    def blk_body(k, carry):
        b = w + NW * k
        pltpu.sync_copy(feat_hbm.at[pl.ds(b * RB * D, RB * D)], rowbuf)
        pltpu.sync_copy(seg_hbm.at[pl.ds(b * RB, RB)], segbuf.at[pl.ds(0, RB)])

        b0 = plsc.load_gather(segbuf, [lax.broadcast(0, (L,))])
        idx0 = (b0 - (b0 & 1)) + iota
        accs = []
        for c in range(NC):
            idxc = idx0 + (c * L)
            old = plsc.load_gather(accf, [idxc])
            a = jnp.maximum(old, rowbuf[pl.ds(c * L, L)])
            plsc.store_scatter(accf, [idxc], a)
            accs.append(a)

        def row_body(r, accs):
            bvec = plsc.load_gather(segbuf, [lax.broadcast(r, (L,))])
            flag = bvec & 1
            newm = flag == 1
            idxb = (bvec - flag) + iota
            off = r * D
            out = []
            for c in range(NC):
                val = rowbuf[pl.ds(off + c * L, L)]
                a = jnp.where(newm, val, jnp.maximum(accs[c], val))
                plsc.store_scatter(accf, [idxb + (c * L)], a)
                out.append(a)
            return tuple(out)

        lax.fori_loop(1, RB, row_body, tuple(accs))
        return carry

    lax.fori_loop(0, nb, blk_body, 0)
    pltpu.sync_copy(accf, part_hbm.at[pl.ds(w * G * D, G * D)])


_SEG_PER_W = G // NW
_CH = _SEG_PER_W * D


@functools.partial(
    pl.kernel,
    out_type=jax.ShapeDtypeStruct((G * D,), jnp.float32),
    mesh=_mesh,
    compiler_params=pltpu.CompilerParams(needs_layout_passes=False),
    scratch_types=[
        pltpu.VMEM((_CH,), jnp.float32),
        pltpu.VMEM((_CH,), jnp.float32),
    ],
)
def _combine(part_hbm, out_hbm, buf, acc):
    w = lax.axis_index("s") * 2 + lax.axis_index("c")
    neg = jnp.full((L,), _NEG, jnp.float32)

    def init_body(i, carry):
        acc[pl.ds(i * L, L)] = neg
        return carry

    lax.fori_loop(0, _CH // L, init_body, 0)

    def t_body(t, carry):
        pltpu.sync_copy(part_hbm.at[pl.ds(t * (G * D) + w * _CH, _CH)], buf)

        def m_body(i, mc):
            sl = pl.ds(i * L, L)
            acc[sl] = jnp.maximum(acc[sl], buf[sl])
            return mc

        lax.fori_loop(0, _CH // L, m_body, 0)
        return carry

    lax.fori_loop(0, NW, t_body, 0)
    pltpu.sync_copy(acc, out_hbm.at[pl.ds(w * _CH, _CH)])


def kernel(feat, segment_ids, num_graphs):
    seg = jnp.minimum(segment_ids, num_graphs - 1).astype(jnp.int32)
    isnew = jnp.concatenate(
        [jnp.ones((1,), jnp.int32), (seg[1:] != seg[:-1]).astype(jnp.int32)]
    )
    aug = seg * jnp.int32(D) + isnew
    part = _seg_max_partial(feat.reshape(-1), aug)
    outf = _combine(part)
    return outf.reshape(G, D)


# speedup vs baseline: 4.1589x; 2.5462x over previous
import functools

import jax
import jax.numpy as jnp
from jax import lax
from jax.experimental import pallas as pl
from jax.experimental.pallas import tpu as pltpu
from jax.experimental.pallas import tpu_sc as plsc

N = 50000
D = 256
G = 256
L = 16
NW = 32
RB = 200
NBLK = N // RB
NC = D // L
_EXTRA = NBLK % NW
_BASE_BLKS = NBLK // NW

_mesh = plsc.VectorSubcoreMesh(core_axis_name="c", subcore_axis_name="s")

_NEG = float("-inf")


@functools.partial(
    pl.kernel,
    out_type=jax.ShapeDtypeStruct((NW, G * D), jnp.float32),
    mesh=_mesh,
    compiler_params=pltpu.CompilerParams(needs_layout_passes=False),
    scratch_types=[
        pltpu.VMEM((RB, D), jnp.float32),
        pltpu.VMEM((G * D,), jnp.float32),
        pltpu.VMEM((256,), jnp.int32),
    ],
)
def _seg_max_partial(feat_hbm, aug_hbm, part_hbm, rowbuf, accf, augbuf):
    w = lax.axis_index("s") * 2 + lax.axis_index("c")
    iota = lax.broadcasted_iota(jnp.int32, (L,), 0)
    neg = jnp.full((L,), _NEG, jnp.float32)

    def init_body(i, carry):
        accf[pl.ds(i * L, L)] = neg
        return carry

    lax.fori_loop(0, G * D // L, init_body, 0)

    nb = jnp.where(w < _EXTRA, _BASE_BLKS + 1, _BASE_BLKS)

    def blk_body(k, carry):
        b = w + NW * k
        pltpu.sync_copy(feat_hbm.at[pl.ds(b * RB, RB), :], rowbuf)
        pltpu.sync_copy(aug_hbm.at[pl.ds(b * RB, RB)], augbuf.at[pl.ds(0, RB)])

        def run_cond(r):
            return r < RB

        def run_body(r):
            v = plsc.load_gather(augbuf, [lax.broadcast(r, (L,))])
            ev = v & 255
            idx0 = (v - ev) + iota
            e = jnp.max(ev)
            accs = []
            for c in range(NC):
                old = plsc.load_gather(accf, [idx0 + (c * L)])
                accs.append(jnp.maximum(old, rowbuf[r, pl.ds(c * L, L)]))

            def row_body(rr, accs):
                return tuple(
                    jnp.maximum(accs[c], rowbuf[rr, pl.ds(c * L, L)])
                    for c in range(NC)
                )

            accs = lax.fori_loop(r + 1, e, row_body, tuple(accs))
            for c in range(NC):
                plsc.store_scatter(accf, [idx0 + (c * L)], accs[c])
            return e

        lax.while_loop(run_cond, run_body, jnp.int32(0))
        return carry

    lax.fori_loop(0, nb, blk_body, 0)
    pltpu.sync_copy(accf, part_hbm.at[w])


_SEG_PER_W = G // NW
_CH = _SEG_PER_W * D


@functools.partial(
    pl.kernel,
    out_type=jax.ShapeDtypeStruct((G * D,), jnp.float32),
    mesh=_mesh,
    compiler_params=pltpu.CompilerParams(needs_layout_passes=False),
    scratch_types=[
        pltpu.VMEM((NW, _CH), jnp.float32),
        pltpu.VMEM((_CH,), jnp.float32),
    ],
)
def _combine(part_hbm, out_hbm, buf, acc):
    w = lax.axis_index("s") * 2 + lax.axis_index("c")
    pltpu.sync_copy(part_hbm.at[:, pl.ds(w * _CH, _CH)], buf)

    def m_body(i, carry):
        sl = pl.ds(i * L, L)
        m = jnp.maximum(buf[0, sl], buf[1, sl])
        for t in range(2, NW):
            m = jnp.maximum(m, buf[t, sl])
        acc[sl] = m
        return carry

    lax.fori_loop(0, _CH // L, m_body, 0)
    pltpu.sync_copy(acc, out_hbm.at[pl.ds(w * _CH, _CH)])


def kernel(feat, segment_ids, num_graphs):
    seg = jnp.minimum(segment_ids, num_graphs - 1).astype(jnp.int32)
    chg = seg[1:] != seg[:-1]
    pos = jnp.where(chg, jnp.arange(1, N, dtype=jnp.int32), jnp.int32(N))
    ends = jnp.concatenate(
        [lax.cummin(pos, axis=0, reverse=True), jnp.full((1,), N, jnp.int32)]
    )
    blk0 = (jnp.arange(N, dtype=jnp.int32) // RB) * RB
    local_end = jnp.minimum(ends, blk0 + RB) - blk0
    aug = seg * jnp.int32(D) + local_end
    part = _seg_max_partial(feat, aug)
    outf = _combine(part)
    return outf.reshape(G, D)


# speedup vs baseline: 4.8548x; 1.1673x over previous
import functools

import jax
import jax.numpy as jnp
from jax import lax
from jax.experimental import pallas as pl
from jax.experimental.pallas import tpu as pltpu
from jax.experimental.pallas import tpu_sc as plsc

N = 50000
D = 256
G = 256
L = 16
NW = 32
RB = 80
NBLK = N // RB
NC = D // L
_EXTRA = NBLK % NW
_BASE_BLKS = NBLK // NW

_mesh = plsc.VectorSubcoreMesh(core_axis_name="c", subcore_axis_name="s")

_NEG = float("-inf")


@functools.partial(
    pl.kernel,
    out_type=jax.ShapeDtypeStruct((NW, G * D), jnp.float32),
    mesh=_mesh,
    compiler_params=pltpu.CompilerParams(needs_layout_passes=False),
    scratch_types=[
        pltpu.VMEM((RB, D), jnp.float32),
        pltpu.VMEM((RB, D), jnp.float32),
        pltpu.VMEM((G * D,), jnp.float32),
        pltpu.VMEM((128,), jnp.int32),
        pltpu.VMEM((128,), jnp.int32),
        pltpu.SemaphoreType.DMA,
        pltpu.SemaphoreType.DMA,
        pltpu.SemaphoreType.DMA,
        pltpu.SemaphoreType.DMA,
    ],
)
def _seg_max_partial(
    feat_hbm, aug_hbm, part_hbm, rowb0, rowb1, accf, augb0, augb1,
    semf0, semf1, sema0, sema1,
):
    w = lax.axis_index("s") * 2 + lax.axis_index("c")
    iota = lax.broadcasted_iota(jnp.int32, (L,), 0)
    neg = jnp.full((L,), _NEG, jnp.float32)
    rowbufs = (rowb0, rowb1)
    augbufs = (augb0, augb1)
    semfs = (semf0, semf1)
    semas = (sema0, sema1)

    def init_body(i, carry):
        accf[pl.ds(i * L, L)] = neg
        return carry

    lax.fori_loop(0, G * D // L, init_body, 0)

    nb = jnp.where(w < _EXTRA, _BASE_BLKS + 1, _BASE_BLKS)

    def start(k, par):
        b = w + NW * k
        pltpu.async_copy(feat_hbm.at[pl.ds(b * RB, RB), :], rowbufs[par], semfs[par])
        pltpu.async_copy(
            aug_hbm.at[pl.ds(b * RB, RB)], augbufs[par].at[pl.ds(0, RB)], semas[par]
        )

    def wait(k, par):
        b = w + NW * k
        pltpu.make_async_copy(
            feat_hbm.at[pl.ds(b * RB, RB), :], rowbufs[par], semfs[par]
        ).wait()
        pltpu.make_async_copy(
            aug_hbm.at[pl.ds(b * RB, RB)], augbufs[par].at[pl.ds(0, RB)], semas[par]
        ).wait()

    def compute(k, par):
        rowbuf = rowbufs[par]
        augbuf = augbufs[par]

        def run_cond(r):
            return r < RB

        def run_body(r):
            v = plsc.load_gather(augbuf, [lax.broadcast(r, (L,))])
            ev = v & 255
            idx0 = (v - ev) + iota
            e = jnp.max(ev)
            accs = []
            for c in range(NC):
                old = plsc.load_gather(accf, [idx0 + (c * L)])
                accs.append(jnp.maximum(old, rowbuf[r, pl.ds(c * L, L)]))

            def row_body(rr, accs):
                return tuple(
                    jnp.maximum(accs[c], rowbuf[rr, pl.ds(c * L, L)])
                    for c in range(NC)
                )

            accs = lax.fori_loop(r + 1, e, row_body, tuple(accs))
            for c in range(NC):
                plsc.store_scatter(accf, [idx0 + (c * L)], accs[c])
            return e

        lax.while_loop(run_cond, run_body, jnp.int32(0))

    start(0, 0)

    @pl.when(nb > 1)
    def _():
        start(1, 1)

    def pair_body(j, carry):
        for par in range(2):
            k = 2 * j + par

            @pl.when(k < nb)
            def _():
                wait(k, par)
                compute(k, par)

                @pl.when(k + 2 < nb)
                def _():
                    start(k + 2, par)

        return carry

    lax.fori_loop(0, (_BASE_BLKS + 2) // 2, pair_body, 0)
    pltpu.sync_copy(accf, part_hbm.at[w])


_SEG_PER_W = G // NW
_CH = _SEG_PER_W * D


@functools.partial(
    pl.kernel,
    out_type=jax.ShapeDtypeStruct((G * D,), jnp.float32),
    mesh=_mesh,
    compiler_params=pltpu.CompilerParams(needs_layout_passes=False),
    scratch_types=[
        pltpu.VMEM((NW, _CH), jnp.float32),
        pltpu.VMEM((_CH,), jnp.float32),
    ],
)
def _combine(part_hbm, out_hbm, buf, acc):
    w = lax.axis_index("s") * 2 + lax.axis_index("c")
    pltpu.sync_copy(part_hbm.at[:, pl.ds(w * _CH, _CH)], buf)

    def m_body(i, carry):
        sl = pl.ds(i * L, L)
        m = jnp.maximum(buf[0, sl], buf[1, sl])
        for t in range(2, NW):
            m = jnp.maximum(m, buf[t, sl])
        acc[sl] = m
        return carry

    lax.fori_loop(0, _CH // L, m_body, 0)
    pltpu.sync_copy(acc, out_hbm.at[pl.ds(w * _CH, _CH)])


def kernel(feat, segment_ids, num_graphs):
    seg = jnp.minimum(segment_ids, num_graphs - 1).astype(jnp.int32)
    chg = seg[1:] != seg[:-1]
    pos = jnp.where(chg, jnp.arange(1, N, dtype=jnp.int32), jnp.int32(N))
    ends = jnp.concatenate(
        [lax.cummin(pos, axis=0, reverse=True), jnp.full((1,), N, jnp.int32)]
    )
    blk0 = (jnp.arange(N, dtype=jnp.int32) // RB) * RB
    local_end = jnp.minimum(ends, blk0 + RB) - blk0
    aug = seg * jnp.int32(D) + local_end
    part = _seg_max_partial(feat, aug)
    outf = _combine(part)
    return outf.reshape(G, D)


# speedup vs baseline: 5.5960x; 1.1527x over previous
import functools

import jax
import jax.numpy as jnp
from jax import lax
from jax.experimental import pallas as pl
from jax.experimental.pallas import tpu as pltpu
from jax.experimental.pallas import tpu_sc as plsc

N = 50000
D = 256
G = 256
L = 16
NW = 32
RB = 80
NBLK = N // RB
NC = D // L
_EXTRA = NBLK % NW
_BASE_BLKS = NBLK // NW

_mesh = plsc.VectorSubcoreMesh(core_axis_name="c", subcore_axis_name="s")

_NEG = float("-inf")


@functools.partial(
    pl.kernel,
    out_type=jax.ShapeDtypeStruct((NW, G * D), jnp.float32),
    mesh=_mesh,
    compiler_params=pltpu.CompilerParams(needs_layout_passes=False),
    scratch_types=[
        pltpu.VMEM((RB, D), jnp.float32),
        pltpu.VMEM((RB, D), jnp.float32),
        pltpu.VMEM((G * D,), jnp.float32),
        pltpu.VMEM((128,), jnp.int32),
        pltpu.VMEM((128,), jnp.int32),
        pltpu.SemaphoreType.DMA,
        pltpu.SemaphoreType.DMA,
        pltpu.SemaphoreType.DMA,
        pltpu.SemaphoreType.DMA,
    ],
)
def _seg_max_partial(
    feat_hbm, aug_hbm, part_hbm, rowb0, rowb1, accf, augb0, augb1,
    semf0, semf1, sema0, sema1,
):
    w = lax.axis_index("s") * 2 + lax.axis_index("c")
    iota = lax.broadcasted_iota(jnp.int32, (L,), 0)
    neg = jnp.full((L,), _NEG, jnp.float32)
    rowbufs = (rowb0, rowb1)
    augbufs = (augb0, augb1)
    semfs = (semf0, semf1)
    semas = (sema0, sema1)

    def init_body(i, carry):
        accf[pl.ds(i * L, L)] = neg
        return carry

    lax.fori_loop(0, G * D // L, init_body, 0)

    for sb in augbufs:
        for i in range(RB // L, 128 // L):
            sb[pl.ds(i * L, L)] = jnp.full((L,), -1, jnp.int32)

    nb = jnp.where(w < _EXTRA, _BASE_BLKS + 1, _BASE_BLKS)

    def start(k, par):
        b = w + NW * k
        pltpu.async_copy(feat_hbm.at[pl.ds(b * RB, RB), :], rowbufs[par], semfs[par])
        pltpu.async_copy(
            aug_hbm.at[pl.ds(b * RB, RB)], augbufs[par].at[pl.ds(0, RB)], semas[par]
        )

    def wait(k, par):
        b = w + NW * k
        pltpu.make_async_copy(
            feat_hbm.at[pl.ds(b * RB, RB), :], rowbufs[par], semfs[par]
        ).wait()
        pltpu.make_async_copy(
            aug_hbm.at[pl.ds(b * RB, RB)], augbufs[par].at[pl.ds(0, RB)], semas[par]
        ).wait()

    def compute(k, par):
        rowbuf = rowbufs[par]
        augbuf = augbufs[par]

        def run_cond(r):
            return r < RB

        def run_body(r):
            sv = plsc.load_gather(augbuf, [lax.broadcast(r, (L,))])
            idx0 = jnp.minimum(sv, G - 1) * D + iota

            def w_cond(c):
                pos, ew = c
                return (ew == L) & (pos < RB)

            def w_body(c):
                pos, _ = c
                wv = augbuf[pl.ds(pos, L)]
                ew = jnp.max(jnp.where(wv == sv, iota + 1, jnp.int32(0)))
                return (jnp.minimum(pos + ew, RB), ew)

            e, _ = lax.while_loop(w_cond, w_body, (r, jnp.int32(L)))
            accs = []
            for c in range(NC):
                old = plsc.load_gather(accf, [idx0 + (c * L)])
                accs.append(jnp.maximum(old, rowbuf[r, pl.ds(c * L, L)]))

            def row_body(rr, accs):
                return tuple(
                    jnp.maximum(accs[c], rowbuf[rr, pl.ds(c * L, L)])
                    for c in range(NC)
                )

            accs = lax.fori_loop(r + 1, e, row_body, tuple(accs))
            for c in range(NC):
                plsc.store_scatter(accf, [idx0 + (c * L)], accs[c])
            return e

        lax.while_loop(run_cond, run_body, jnp.int32(0))

    start(0, 0)

    @pl.when(nb > 1)
    def _():
        start(1, 1)

    def pair_body(j, carry):
        for par in range(2):
            k = 2 * j + par

            @pl.when(k < nb)
            def _():
                wait(k, par)
                compute(k, par)

                @pl.when(k + 2 < nb)
                def _():
                    start(k + 2, par)

        return carry

    lax.fori_loop(0, (_BASE_BLKS + 2) // 2, pair_body, 0)
    pltpu.sync_copy(accf, part_hbm.at[w])


_SEG_PER_W = G // NW
_CH = _SEG_PER_W * D


@functools.partial(
    pl.kernel,
    out_type=jax.ShapeDtypeStruct((G * D,), jnp.float32),
    mesh=_mesh,
    compiler_params=pltpu.CompilerParams(needs_layout_passes=False),
    scratch_types=[
        pltpu.VMEM((NW, _CH), jnp.float32),
        pltpu.VMEM((_CH,), jnp.float32),
    ],
)
def _combine(part_hbm, out_hbm, buf, acc):
    w = lax.axis_index("s") * 2 + lax.axis_index("c")
    pltpu.sync_copy(part_hbm.at[:, pl.ds(w * _CH, _CH)], buf)

    def m_body(i, carry):
        sl = pl.ds(i * L, L)
        m = jnp.maximum(buf[0, sl], buf[1, sl])
        for t in range(2, NW):
            m = jnp.maximum(m, buf[t, sl])
        acc[sl] = m
        return carry

    lax.fori_loop(0, _CH // L, m_body, 0)
    pltpu.sync_copy(acc, out_hbm.at[pl.ds(w * _CH, _CH)])


def kernel(feat, segment_ids, num_graphs):
    part = _seg_max_partial(feat, segment_ids.astype(jnp.int32))
    outf = _combine(part)
    return outf.reshape(G, D)


# speedup vs baseline: 7.3789x; 1.3186x over previous
import functools

import jax
import jax.numpy as jnp
from jax import lax
from jax.experimental import pallas as pl
from jax.experimental.pallas import tpu as pltpu
from jax.experimental.pallas import tpu_sc as plsc

N = 50000
D = 256
G = 256
L = 16
NW = 32
RB = 80
NBLK = N // RB
NC = D // L
_EXTRA = NBLK % NW
_BASE_BLKS = NBLK // NW

_mesh = plsc.VectorSubcoreMesh(core_axis_name="c", subcore_axis_name="s")

_NEG = float("-inf")


@functools.partial(
    pl.kernel,
    out_type=jax.ShapeDtypeStruct((NW, G * D), jnp.float32),
    mesh=_mesh,
    compiler_params=pltpu.CompilerParams(needs_layout_passes=False),
    scratch_types=[
        pltpu.VMEM((RB, D), jnp.float32),
        pltpu.VMEM((RB, D), jnp.float32),
        pltpu.VMEM((RB, D), jnp.float32),
        pltpu.VMEM((G * D,), jnp.float32),
        pltpu.VMEM((128,), jnp.int32),
        pltpu.VMEM((128,), jnp.int32),
        pltpu.VMEM((128,), jnp.int32),
        pltpu.SemaphoreType.DMA,
        pltpu.SemaphoreType.DMA,
        pltpu.SemaphoreType.DMA,
        pltpu.SemaphoreType.DMA,
        pltpu.SemaphoreType.DMA,
        pltpu.SemaphoreType.DMA,
    ],
)
def _seg_max_partial(
    feat_hbm, aug_hbm, part_hbm, rowb0, rowb1, rowb2, accf, augb0, augb1,
    augb2, semf0, semf1, semf2, sema0, sema1, sema2,
):
    w = lax.axis_index("s") * 2 + lax.axis_index("c")
    iota = lax.broadcasted_iota(jnp.int32, (L,), 0)
    neg = jnp.full((L,), _NEG, jnp.float32)
    rowbufs = (rowb0, rowb1, rowb2)
    augbufs = (augb0, augb1, augb2)
    semfs = (semf0, semf1, semf2)
    semas = (sema0, sema1, sema2)

    def init_body(i, carry):
        for u in range(4):
            accf[pl.ds((4 * i + u) * L, L)] = neg
        return carry

    lax.fori_loop(0, G * D // L // 4, init_body, 0)

    pad = jnp.full((L,), -1, jnp.int32)
    for sb in augbufs:
        for i in range(RB // L, 128 // L):
            sb[pl.ds(i * L, L)] = pad

    nb = jnp.where(w < _EXTRA, _BASE_BLKS + 1, _BASE_BLKS)

    def start(k, par):
        b = w + NW * k
        pltpu.async_copy(feat_hbm.at[pl.ds(b * RB, RB), :], rowbufs[par], semfs[par])
        pltpu.async_copy(
            aug_hbm.at[pl.ds(b * RB, RB)], augbufs[par].at[pl.ds(0, RB)], semas[par]
        )

    def wait(k, par):
        b = w + NW * k
        pltpu.make_async_copy(
            feat_hbm.at[pl.ds(b * RB, RB), :], rowbufs[par], semfs[par]
        ).wait()
        pltpu.make_async_copy(
            aug_hbm.at[pl.ds(b * RB, RB)], augbufs[par].at[pl.ds(0, RB)], semas[par]
        ).wait()

    def compute(k, par):
        rowbuf = rowbufs[par]
        augbuf = augbufs[par]

        def run_cond(r):
            return r < RB

        def run_body(r):
            sv = plsc.load_gather(augbuf, [lax.broadcast(r, (L,))])
            idx0 = jnp.minimum(sv, G - 1) * D + iota

            def w_cond(c):
                pos, ew = c
                return (ew == L) & (pos < RB)

            def w_body(c):
                pos, _ = c
                wv = augbuf[pl.ds(pos, L)]
                ew = jnp.max(jnp.where(wv == sv, iota + 1, jnp.int32(0)))
                return (jnp.minimum(pos + ew, RB), ew)

            e, _ = lax.while_loop(w_cond, w_body, (r, jnp.int32(L)))
            accs = []
            for c in range(NC):
                old = plsc.load_gather(accf, [idx0 + (c * L)])
                accs.append(jnp.maximum(old, rowbuf[r, pl.ds(c * L, L)]))

            def row_body(rr, accs):
                return tuple(
                    jnp.maximum(accs[c], rowbuf[rr, pl.ds(c * L, L)])
                    for c in range(NC)
                )

            accs = lax.fori_loop(r + 1, e, row_body, tuple(accs))
            for c in range(NC):
                plsc.store_scatter(accf, [idx0 + (c * L)], accs[c])
            return e

        lax.while_loop(run_cond, run_body, jnp.int32(0))

    start(0, 0)

    @pl.when(nb > 1)
    def _():
        start(1, 1)

    @pl.when(nb > 2)
    def _():
        start(2, 2)

    def trio_body(j, carry):
        for par in range(3):
            k = 3 * j + par

            @pl.when(k < nb)
            def _():
                wait(k, par)
                compute(k, par)

                @pl.when(k + 3 < nb)
                def _():
                    start(k + 3, par)

        return carry

    lax.fori_loop(0, (_BASE_BLKS + 3) // 3, trio_body, 0)
    pltpu.sync_copy(accf, part_hbm.at[w])


_SEG_PER_W = G // NW
_CH = _SEG_PER_W * D


@functools.partial(
    pl.kernel,
    out_type=jax.ShapeDtypeStruct((G * D,), jnp.float32),
    mesh=_mesh,
    compiler_params=pltpu.CompilerParams(needs_layout_passes=False),
    scratch_types=[
        pltpu.VMEM((NW, _CH), jnp.float32),
        pltpu.VMEM((_CH,), jnp.float32),
    ],
)
def _combine(part_hbm, out_hbm, buf, acc):
    w = lax.axis_index("s") * 2 + lax.axis_index("c")
    pltpu.sync_copy(part_hbm.at[:, pl.ds(w * _CH, _CH)], buf)

    def m_body(i, carry):
        sl = pl.ds(i * L, L)
        m = jnp.maximum(buf[0, sl], buf[1, sl])
        for t in range(2, NW):
            m = jnp.maximum(m, buf[t, sl])
        acc[sl] = m
        return carry

    lax.fori_loop(0, _CH // L, m_body, 0)
    pltpu.sync_copy(acc, out_hbm.at[pl.ds(w * _CH, _CH)])


def kernel(feat, segment_ids, num_graphs):
    part = _seg_max_partial(feat, segment_ids.astype(jnp.int32))
    outf = _combine(part)
    return outf.reshape(G, D)
